# baseline (speedup 1.0000x reference)
import functools

import jax
import jax.numpy as jnp
from jax import lax
from jax.experimental import pallas as pl
from jax.experimental.pallas import tpu as pltpu
from jax.experimental.pallas import tpu_sc as plsc

VOCAB = 100000
HD = 64
HASH = 4096
B = 4096
L = 50
LP = 56
NC = 2
NS = 16
NW = NC * NS
RPW = B // NW
NB = RPW // 2
GI = 2 * LP
NBUF = 4
MULT = VOCAB % HASH
MASK = HASH - 1

_mesh = plsc.VectorSubcoreMesh(core_axis_name="c", subcore_axis_name="s")


@functools.partial(
    pl.kernel,
    out_type=jax.ShapeDtypeStruct((B, L, 2 * HD), jnp.float32),
    mesh=_mesh,
    compiler_params=pltpu.CompilerParams(needs_layout_passes=False),
    scratch_types=[
        pltpu.VMEM((RPW, L), jnp.int32),
        pltpu.VMEM((NB, GI), jnp.int32),
        pltpu.VMEM((NB, GI), jnp.int32),
        pltpu.VMEM((NBUF, GI, 2 * HD), jnp.float32),
        pltpu.SemaphoreType.DMA((NBUF,)),
        pltpu.SemaphoreType.DMA((NBUF,)),
        pltpu.SemaphoreType.DMA((NBUF,)),
    ],
)
def _bigram_gather(ids_hbm, u_hbm, b_hbm, out_hbm,
                   ids_v, ui_v, bi_v, comb_v, sem_u, sem_b, sem_w):
    wid = lax.axis_index("s") * NC + lax.axis_index("c")
    rbase = wid * RPW
    pltpu.sync_copy(ids_hbm.at[pl.ds(rbase, RPW)], ids_v)

    lanes = lax.iota(jnp.int32, 16)
    zeros16 = jnp.zeros((16,), jnp.int32)
    pl1 = jnp.maximum(lanes - 1, 0)

    def fill_body(r, carry):
        g = r // 2
        cb = (r & 1) * LP
        ui_v[g, pl.ds(cb + 40, 16)] = zeros16
        bi_v[g, pl.ds(cb + 40, 16)] = zeros16
        rvec = r + zeros16
        for c, co in ((0, 0), (1, 16), (2, 32), (3, 34)):
            cur = ids_v[r, pl.ds(co, 16)]
            if c == 0:
                prev0 = plsc.load_gather(ids_v, [rvec, pl1])
                pi = jnp.where(lanes == 0, 0, prev0)
            else:
                pi = ids_v[r, pl.ds(co - 1, 16)]
            ui_v[g, pl.ds(cb + co, 16)] = cur
            bi_v[g, pl.ds(cb + co, 16)] = ((pi & MASK) * MULT + cur) & MASK
        return carry

    lax.fori_loop(0, RPW, fill_body, 0)

    def issue_u(g, n):
        pltpu.async_copy(u_hbm.at[ui_v.at[g]], comb_v.at[n], sem_u.at[n])

    def issue_add(g, n):
        pltpu.async_copy(b_hbm.at[bi_v.at[g]], comb_v.at[n], sem_b.at[n],
                         add=True)

    def issue_w(g, n):
        rb = rbase + 2 * g
        pltpu.async_copy(comb_v.at[n, pl.ds(0, L)], out_hbm.at[rb],
                         sem_w.at[n])
        pltpu.async_copy(comb_v.at[n, pl.ds(LP, L)], out_hbm.at[rb + 1],
                         sem_w.at[n])

    def wait_u(n):
        pltpu.make_async_copy(u_hbm.at[pl.ds(0, GI)], comb_v.at[n],
                              sem_u.at[n]).wait()

    def wait_b(n):
        pltpu.make_async_copy(b_hbm.at[pl.ds(0, GI)], comb_v.at[n],
                              sem_b.at[n]).wait()

    def wait_w(n):
        for _ in range(2):
            pltpu.make_async_copy(comb_v.at[n, pl.ds(0, L)],
                                  out_hbm.at[rbase], sem_w.at[n]).wait()

    def pipe_body(i, carry):
        for k in range(NBUF):
            g = NBUF * i + k

            @pl.when(jnp.logical_and(g >= NBUF, g < NB))
            def _():
                wait_w(k)

            @pl.when(g < NB)
            def _():
                issue_u(g, k)

            g1 = g - 1
            k1 = (k - 1) % NBUF

            @pl.when(jnp.logical_and(g1 >= 0, g1 < NB))
            def _():
                wait_u(k1)
                issue_add(g1, k1)

            g2 = g - 2
            k2 = (k - 2) % NBUF

            @pl.when(jnp.logical_and(g2 >= 0, g2 < NB))
            def _():
                wait_b(k2)
                issue_w(g2, k2)

        return carry

    lax.fori_loop(0, (NB + 2 + NBUF - 1) // NBUF + 1, pipe_body, 0)

    for n in range(NBUF):
        wait_w(n)


def kernel(ids, u, b):
    u128 = jnp.pad(u, ((0, 0), (0, HD)))
    b128 = jnp.pad(b, ((0, 0), (HD, 0)))
    return _bigram_gather(ids, u128, b128)


# speedup vs baseline: 3.8622x; 3.8622x over previous
import functools

import jax
import jax.numpy as jnp
from jax import lax
from jax.experimental import pallas as pl
from jax.experimental.pallas import tpu as pltpu
from jax.experimental.pallas import tpu_sc as plsc

VOCAB = 100000
HD = 64
HASH = 4096
B = 4096
L = 50
N = B * L
NC = 2
NS = 16
NW = NC * NS
CHUNK = N // NW
GB = 128
NG = CHUNK // GB
MULT = VOCAB % HASH
MASK = HASH - 1

_mesh = plsc.VectorSubcoreMesh(core_axis_name="c", subcore_axis_name="s")


@functools.partial(
    pl.kernel,
    out_type=jax.ShapeDtypeStruct((N, 2 * HD), jnp.float32),
    mesh=_mesh,
    compiler_params=pltpu.CompilerParams(use_tc_tiling_on_sc=False),
    scratch_types=[
        pltpu.VMEM((CHUNK + 16,), jnp.int32),
        pltpu.VMEM((CHUNK,), jnp.int32),
        pltpu.VMEM((2, GB, HD), jnp.float32),
        pltpu.VMEM((2, GB, HD), jnp.float32),
        pltpu.SemaphoreType.DMA((2,)),
        pltpu.SemaphoreType.DMA((2,)),
    ],
)
def _bigram_gather(ids_hbm, u_hbm, b_hbm, out_hbm,
                   ids_v, bi_v, ue_v, be_v, sem_u, sem_b):
    wid = lax.axis_index("s") * NC + lax.axis_index("c")
    base = wid * CHUNK
    pltpu.sync_copy(ids_hbm.at[pl.ds(base, CHUNK)], ids_v.at[pl.ds(16, CHUNK)])

    lanes = lax.iota(jnp.int32, 16)

    def bi_body(j, carry):
        o = 16 + j * 16
        cur = ids_v[pl.ds(o, 16)]
        prev = ids_v[pl.ds(o - 1, 16)]
        pos = (j * 16 + lanes) % L
        pi = jnp.where(pos == 0, 0, prev)
        bi_v[pl.ds(j * 16, 16)] = ((pi & MASK) * MULT + cur) & MASK
        return carry

    lax.fori_loop(0, CHUNK // 16, bi_body, 0)

    def issue(g, p):
        pltpu.async_copy(u_hbm.at[ids_v.at[pl.ds(16 + g * GB, GB)]],
                         ue_v.at[p], sem_u.at[p])
        pltpu.async_copy(b_hbm.at[bi_v.at[pl.ds(g * GB, GB)]],
                         be_v.at[p], sem_b.at[p])

    def wait_gathers(p):
        pltpu.make_async_copy(u_hbm.at[pl.ds(0, GB)], ue_v.at[p],
                              sem_u.at[p]).wait()
        pltpu.make_async_copy(b_hbm.at[pl.ds(0, GB)], be_v.at[p],
                              sem_b.at[p]).wait()

    issue(0, 0)

    def g_body(i, carry):
        for (off, p, q) in ((0, 0, 1), (1, 1, 0)):
            g = 2 * i + off

            @pl.when(g + 1 < NG)
            def _():
                issue(g + 1, q)

            wait_gathers(p)
            ro = base + g * GB
            pltpu.sync_copy(ue_v.at[p], out_hbm.at[pl.ds(ro, GB), pl.ds(0, HD)])
            pltpu.sync_copy(be_v.at[p], out_hbm.at[pl.ds(ro, GB), pl.ds(HD, HD)])
        return carry

    lax.fori_loop(0, NG // 2, g_body, 0)


def kernel(ids, u, b):
    out = _bigram_gather(ids.reshape(N), u, b)
    return out.reshape(B, L, 2 * HD)


# speedup vs baseline: 3.8658x; 1.0009x over previous
import functools

import jax
import jax.numpy as jnp
from jax import lax
from jax.experimental import pallas as pl
from jax.experimental.pallas import tpu as pltpu
from jax.experimental.pallas import tpu_sc as plsc

VOCAB = 100000
HD = 64
HASH = 4096
B = 4096
L = 50
N = B * L
NC = 2
NS = 16
NW = NC * NS
CHUNK = N // NW
GB = 128
NG = CHUNK // GB
MULT = VOCAB % HASH
MASK = HASH - 1

_mesh = plsc.VectorSubcoreMesh(core_axis_name="c", subcore_axis_name="s")


@functools.partial(
    pl.kernel,
    out_type=jax.ShapeDtypeStruct((N, 2 * HD), jnp.float32),
    mesh=_mesh,
    compiler_params=pltpu.CompilerParams(use_tc_tiling_on_sc=False),
    scratch_types=[
        pltpu.VMEM((CHUNK + 16,), jnp.int32),
        pltpu.VMEM((CHUNK,), jnp.int32),
        pltpu.VMEM((2, GB, HD), jnp.float32),
        pltpu.VMEM((2, GB, HD), jnp.float32),
        pltpu.SemaphoreType.DMA((2,)),
        pltpu.SemaphoreType.DMA((2,)),
        pltpu.SemaphoreType.DMA((2,)),
    ],
)
def _bigram_gather(ids_hbm, u_hbm, b_hbm, out_hbm,
                   ids_v, bi_v, ue_v, be_v, sem_u, sem_b, sem_w):
    wid = lax.axis_index("s") * NC + lax.axis_index("c")
    base = wid * CHUNK
    pltpu.sync_copy(ids_hbm.at[pl.ds(base, CHUNK)], ids_v.at[pl.ds(16, CHUNK)])

    lanes = lax.iota(jnp.int32, 16)

    def bi_body(j, carry):
        o = 16 + j * 16
        cur = ids_v[pl.ds(o, 16)]
        prev = ids_v[pl.ds(o - 1, 16)]
        pos = (j * 16 + lanes) % L
        pi = jnp.where(pos == 0, 0, prev)
        bi_v[pl.ds(j * 16, 16)] = ((pi & MASK) * MULT + cur) & MASK
        return carry

    lax.fori_loop(0, CHUNK // 16, bi_body, 0)

    def issue(g, p):
        pltpu.async_copy(u_hbm.at[ids_v.at[pl.ds(16 + g * GB, GB)]],
                         ue_v.at[p], sem_u.at[p])
        pltpu.async_copy(b_hbm.at[bi_v.at[pl.ds(g * GB, GB)]],
                         be_v.at[p], sem_b.at[p])

    def wait_gathers(p):
        pltpu.make_async_copy(u_hbm.at[pl.ds(0, GB)], ue_v.at[p],
                              sem_u.at[p]).wait()
        pltpu.make_async_copy(b_hbm.at[pl.ds(0, GB)], be_v.at[p],
                              sem_b.at[p]).wait()

    def issue_w(g, p):
        ro = base + g * GB
        pltpu.async_copy(ue_v.at[p], out_hbm.at[pl.ds(ro, GB), pl.ds(0, HD)],
                         sem_w.at[p])
        pltpu.async_copy(be_v.at[p], out_hbm.at[pl.ds(ro, GB), pl.ds(HD, HD)],
                         sem_w.at[p])

    def wait_w(p):
        pltpu.make_async_copy(ue_v.at[p],
                              out_hbm.at[pl.ds(base, GB), pl.ds(0, HD)],
                              sem_w.at[p]).wait()
        pltpu.make_async_copy(be_v.at[p],
                              out_hbm.at[pl.ds(base, GB), pl.ds(HD, HD)],
                              sem_w.at[p]).wait()

    issue(0, 0)

    def g_body(i, carry):
        for (off, p, q) in ((0, 0, 1), (1, 1, 0)):
            g = 2 * i + off

            @pl.when(g + 1 < NG)
            def _():
                @pl.when(g >= 1)
                def _():
                    wait_w(q)

                issue(g + 1, q)

            wait_gathers(p)
            issue_w(g, p)
        return carry

    lax.fori_loop(0, NG // 2, g_body, 0)
    wait_w(0)
    wait_w(1)


def kernel(ids, u, b):
    out = _bigram_gather(ids.reshape(N), u, b)
    return out.reshape(B, L, 2 * HD)
